# baseline (device time: 8021 ns/iter reference)
import jax
import jax.numpy as jnp
from jax import lax
from jax.experimental import pallas as pl
from jax.experimental.pallas import tpu as pltpu

N_DEV = 4
CHUNKS = 8


def kernel(x):
    m_per, n = x.shape
    rows_per = m_per // CHUNKS
    inv_count = 1.0 / (N_DEV * m_per)

    def body(x_ref, out_ref, acc_ref, comm_ref, send_sems, recv_sems):
        i = pl.program_id(0)
        my_pos = lax.axis_index("i")
        barrier_sem = pltpu.get_barrier_semaphore()

        @pl.when(i == 0)
        def _():
            for k in range(1, N_DEV):
                peer = lax.rem(my_pos + k, N_DEV)
                pl.semaphore_signal(
                    barrier_sem, inc=1,
                    device_id=(peer,), device_id_type=pl.DeviceIdType.MESH,
                )

        chunk_sum = jnp.sum(x_ref[:, :], axis=0, keepdims=True)

        @pl.when(i == 0)
        def _():
            acc_ref[:, :] = chunk_sum

        @pl.when(i > 0)
        def _():
            acc_ref[:, :] = acc_ref[:, :] + chunk_sum

        @pl.when(i == CHUNKS - 1)
        def _():
            pl.semaphore_wait(barrier_sem, N_DEV - 1)
            rdmas = []
            for k in range(1, N_DEV):
                peer = lax.rem(my_pos + k, N_DEV)
                rdma = pltpu.make_async_remote_copy(
                    src_ref=acc_ref,
                    dst_ref=comm_ref.at[k - 1],
                    send_sem=send_sems.at[k - 1],
                    recv_sem=recv_sems.at[k - 1],
                    device_id=(peer,),
                    device_id_type=pl.DeviceIdType.MESH,
                )
                rdma.start()
                rdmas.append(rdma)

            total = acc_ref[:, :]
            for k in range(1, N_DEV):
                rdmas[k - 1].wait_recv()
                total = total + comm_ref[k - 1, :, :]
            out_ref[:, :] = total * inv_count

            for k in range(1, N_DEV):
                rdmas[k - 1].wait_send()

    return pl.pallas_call(
        body,
        grid=(CHUNKS,),
        out_shape=jax.ShapeDtypeStruct((1, n), jnp.float32),
        in_specs=[pl.BlockSpec((rows_per, n), lambda i: (i, 0))],
        out_specs=pl.BlockSpec((1, n), lambda i: (0, 0)),
        scratch_shapes=[
            pltpu.VMEM((1, n), jnp.float32),
            pltpu.VMEM((N_DEV - 1, 1, n), jnp.float32),
            pltpu.SemaphoreType.DMA((N_DEV - 1,)),
            pltpu.SemaphoreType.DMA((N_DEV - 1,)),
        ],
        compiler_params=pltpu.CompilerParams(
            collective_id=0,
            dimension_semantics=("arbitrary",),
        ),
    )(x)


# device time: 7848 ns/iter; 1.0220x vs baseline; 1.0220x over previous
import jax
import jax.numpy as jnp
from jax import lax
from jax.experimental import pallas as pl
from jax.experimental.pallas import tpu as pltpu

N_DEV = 4


def kernel(x):
    m_per, n = x.shape
    inv_count = 1.0 / (N_DEV * m_per)

    def body(x_ref, out_ref, partial_ref, comm_ref, send_sems, recv_sems):
        my_pos = lax.axis_index("i")

        barrier_sem = pltpu.get_barrier_semaphore()
        for k in range(1, N_DEV):
            peer = lax.rem(my_pos + k, N_DEV)
            pl.semaphore_signal(
                barrier_sem, inc=1,
                device_id=(peer,), device_id_type=pl.DeviceIdType.MESH,
            )

        partial_ref[:, :] = jnp.sum(x_ref[:, :], axis=0, keepdims=True)

        pl.semaphore_wait(barrier_sem, N_DEV - 1)

        rdmas = []
        for k in range(1, N_DEV):
            peer = lax.rem(my_pos + k, N_DEV)
            rdma = pltpu.make_async_remote_copy(
                src_ref=partial_ref,
                dst_ref=comm_ref.at[k - 1],
                send_sem=send_sems.at[k - 1],
                recv_sem=recv_sems.at[k - 1],
                device_id=(peer,),
                device_id_type=pl.DeviceIdType.MESH,
            )
            rdma.start()
            rdmas.append(rdma)

        acc = partial_ref[:, :]
        for k in range(1, N_DEV):
            rdmas[k - 1].wait_recv()
            acc = acc + comm_ref[k - 1, :, :]
        out_ref[:, :] = acc * inv_count

        for k in range(1, N_DEV):
            rdmas[k - 1].wait_send()

    return pl.pallas_call(
        body,
        out_shape=jax.ShapeDtypeStruct((1, n), jnp.float32),
        in_specs=[pl.BlockSpec(memory_space=pltpu.VMEM)],
        out_specs=pl.BlockSpec(memory_space=pltpu.VMEM),
        scratch_shapes=[
            pltpu.VMEM((1, n), jnp.float32),
            pltpu.VMEM((N_DEV - 1, 1, n), jnp.float32),
            pltpu.SemaphoreType.DMA((N_DEV - 1,)),
            pltpu.SemaphoreType.DMA((N_DEV - 1,)),
        ],
        compiler_params=pltpu.CompilerParams(collective_id=0),
    )(x)
